# baseline (speedup 1.0000x reference)
import functools

import jax
import jax.numpy as jnp
from jax import lax
from jax.experimental import pallas as pl
from jax.experimental.pallas import tpu as pltpu
from jax.experimental.pallas import tpu_sc as plsc


@functools.lru_cache(maxsize=None)
def _build_sc_kernel(V, D, L, R, C, K):
    NCB = L // C
    NRB = L // R
    W = C + R

    info = plsc.get_sparse_core_info()
    num_workers = info.num_cores * info.num_subcores
    tiles = NRB * NCB
    assert tiles % num_workers == 0
    tiles_per_worker = tiles // num_workers

    mesh = plsc.VectorSubcoreMesh(core_axis_name="c", subcore_axis_name="s")

    @functools.partial(
        pl.kernel,
        out_type=jax.ShapeDtypeStruct((L, L, D), jnp.float32),
        name="toeplitz_gather_sc",
        mesh=mesh,
        scratch_types=[
            pltpu.VMEM((W, D), jnp.float32),
            pltpu.SemaphoreType.DMA,
        ],
    )
    def sc_kernel(table, out, win, sem):
        wid = lax.axis_index("s") * info.num_cores + lax.axis_index("c")

        def tile_body(t, carry):
            tid = wid * tiles_per_worker + t
            rb = tid // NCB
            cb = tid % NCB
            i0 = rb * R
            j0 = cb * C
            base = pl.multiple_of((L - 1) + j0 - i0 - (R - 1), 8)
            pltpu.sync_copy(table.at[pl.ds(base, W)], win)

            def fire(r, c):
                pltpu.async_copy(
                    win.at[pl.ds(R - 1 - r, C)],
                    out.at[i0 + r, pl.ds(pl.multiple_of(j0, 8), C)],
                    sem,
                )
                return c

            def wait_one(r, c):
                pltpu.make_async_copy(
                    win.at[pl.ds(0, C)], out.at[0, pl.ds(0, C)], sem
                ).wait()
                return c

            def steady(r, c):
                c = wait_one(r, c)
                return fire(r, c)

            carry = lax.fori_loop(0, K, fire, carry)
            carry = lax.fori_loop(K, R, steady, carry)
            carry = lax.fori_loop(0, K, wait_one, carry)
            return carry

        lax.fori_loop(0, tiles_per_worker, tile_body, 0)

    return sc_kernel


@functools.lru_cache(maxsize=None)
def _build_sc_spmem_kernel(Vpad, D, L, K):
    info = plsc.get_sparse_core_info()
    num_workers = info.num_cores * info.num_subcores
    rows_per_worker = L // num_workers
    rows_per_stager = Vpad // info.num_subcores

    mesh = plsc.VectorSubcoreMesh(core_axis_name="c", subcore_axis_name="s")

    @functools.partial(
        pl.kernel,
        out_type=jax.ShapeDtypeStruct((L, L, D), jnp.float32),
        name="toeplitz_gather_sc_spmem",
        mesh=mesh,
        scratch_types=[
            pltpu.VMEM_SHARED((Vpad, D), jnp.float32),
            pltpu.SemaphoreType.DMA,
        ],
    )
    def sc_kernel(table, out, shared, sem):
        cid = lax.axis_index("c")
        sid = lax.axis_index("s")
        wid = sid * info.num_cores + cid
        s0 = pl.multiple_of(sid * rows_per_stager, 8)
        pltpu.sync_copy(table.at[pl.ds(s0, rows_per_stager)],
                        shared.at[pl.ds(s0, rows_per_stager)])
        plsc.subcore_barrier()

        i0 = wid * rows_per_worker

        def fire(r, c):
            i = i0 + r
            pltpu.async_copy(
                shared.at[pl.ds(L - 1 - i, L)],
                out.at[i],
                sem,
            )
            return c

        def wait_one(r, c):
            pltpu.make_async_copy(
                shared.at[pl.ds(0, L)], out.at[0], sem
            ).wait()
            return c

        def steady(r, c):
            return fire(r, wait_one(r, c))

        c = lax.fori_loop(0, K, fire, 0)
        c = lax.fori_loop(K, rows_per_worker, steady, c)
        lax.fori_loop(0, K, wait_one, c)

    return sc_kernel


def kernel(rel_emb, length):
    V, D = rel_emb.shape
    L = (V + 1) // 2
    table = jnp.concatenate([rel_emb, jnp.zeros((1, D), rel_emb.dtype)], axis=0)
    return _build_sc_spmem_kernel(V + 1, D, L, 4)(table)


# speedup vs baseline: 1.1982x; 1.1982x over previous
import functools

import jax
import jax.numpy as jnp
from jax import lax
from jax.experimental import pallas as pl
from jax.experimental.pallas import tpu as pltpu
from jax.experimental.pallas import tpu_sc as plsc


@functools.lru_cache(maxsize=None)
def _build_sc_kernel(V, D, L, R, C, K):
    NCB = L // C
    NRB = L // R
    W = C + R

    info = plsc.get_sparse_core_info()
    num_workers = info.num_cores * info.num_subcores
    tiles = NRB * NCB
    assert tiles % num_workers == 0
    tiles_per_worker = tiles // num_workers

    mesh = plsc.VectorSubcoreMesh(core_axis_name="c", subcore_axis_name="s")

    @functools.partial(
        pl.kernel,
        out_type=jax.ShapeDtypeStruct((L * L, D), jnp.float32),
        name="toeplitz_gather_sc",
        mesh=mesh,
        scratch_types=[
            pltpu.VMEM((W, D), jnp.float32),
            pltpu.SemaphoreType.DMA,
        ],
    )
    def sc_kernel(table, out, win, sem):
        wid = lax.axis_index("s") * info.num_cores + lax.axis_index("c")

        def tile_body(t, carry):
            tid = wid * tiles_per_worker + t
            rb = tid // NCB
            cb = tid % NCB
            i0 = rb * R
            j0 = cb * C
            base = pl.multiple_of((L - 1) + j0 - i0 - (R - 1), 8)
            pltpu.sync_copy(table.at[pl.ds(base, W)], win)

            def fire(r, c):
                pltpu.async_copy(
                    win.at[pl.ds(R - 1 - r, C)],
                    out.at[pl.ds(pl.multiple_of((i0 + r) * L + j0, 8), C)],
                    sem,
                )
                return c

            def wait_one(r, c):
                pltpu.make_async_copy(
                    win.at[pl.ds(0, C)], out.at[pl.ds(0, C)], sem
                ).wait()
                return c

            def steady(r, c):
                c = wait_one(r, c)
                return fire(r, c)

            carry = lax.fori_loop(0, K, fire, carry)
            carry = lax.fori_loop(K, R, steady, carry)
            carry = lax.fori_loop(0, K, wait_one, carry)
            return carry

        lax.fori_loop(0, tiles_per_worker, tile_body, 0)

    return sc_kernel


@functools.lru_cache(maxsize=None)
def _build_sc_kernel_part(Vpad, D, L, R, C, K, row_off, nrows):
    NCB = L // C
    NRB = nrows // R
    W = C + R

    info = plsc.get_sparse_core_info()
    num_workers = info.num_cores * info.num_subcores
    assert NRB * NCB == num_workers

    mesh = plsc.VectorSubcoreMesh(core_axis_name="c", subcore_axis_name="s")

    @functools.partial(
        pl.kernel,
        out_type=jax.ShapeDtypeStruct((nrows * L, D), jnp.float32),
        name=f"toeplitz_gather_sc_p{row_off}",
        mesh=mesh,
        scratch_types=[
            pltpu.VMEM((W, D), jnp.float32),
            pltpu.SemaphoreType.DMA,
        ],
    )
    def sc_kernel(table, out, win, sem):
        wid = lax.axis_index("s") * info.num_cores + lax.axis_index("c")
        rb = wid // NCB
        cb = wid % NCB
        i0 = rb * R
        j0 = cb * C
        base = pl.multiple_of((L - 1) + j0 - (row_off + i0) - (R - 1), 8)
        pltpu.sync_copy(table.at[pl.ds(base, W)], win)

        def fire(r, c):
            pltpu.async_copy(
                win.at[pl.ds(R - 1 - r, C)],
                out.at[pl.ds(pl.multiple_of((i0 + r) * L + j0, 8), C)],
                sem,
            )
            return c

        def wait_one(r, c):
            pltpu.make_async_copy(
                win.at[pl.ds(0, C)], out.at[pl.ds(0, C)], sem
            ).wait()
            return c

        def steady(r, c):
            return fire(r, wait_one(r, c))

        c = lax.fori_loop(0, K, fire, 0)
        c = lax.fori_loop(K, R, steady, c)
        lax.fori_loop(0, K, wait_one, c)

    return sc_kernel


@functools.lru_cache(maxsize=None)
def _build_sc_spmem_kernel(Vpad, D, L, K):
    info = plsc.get_sparse_core_info()
    num_workers = info.num_cores * info.num_subcores
    rows_per_worker = L // num_workers
    rows_per_stager = Vpad // info.num_subcores

    mesh = plsc.VectorSubcoreMesh(core_axis_name="c", subcore_axis_name="s")

    @functools.partial(
        pl.kernel,
        out_type=jax.ShapeDtypeStruct((L * L, D), jnp.float32),
        name="toeplitz_gather_sc_spmem",
        mesh=mesh,
        scratch_types=[
            pltpu.VMEM((768, D), jnp.float32),
            pltpu.VMEM_SHARED((Vpad, D), jnp.float32),
            pltpu.SemaphoreType.DMA,
        ],
    )
    def sc_kernel(table, out, win, shared, sem):
        cid = lax.axis_index("c")
        sid = lax.axis_index("s")
        wid = sid * info.num_cores + cid
        pltpu.sync_copy(table.at[pl.ds(0, 768)], win)
        s0 = pl.multiple_of(sid * rows_per_stager, 8)
        pltpu.sync_copy(table.at[pl.ds(s0, rows_per_stager)],
                        shared.at[pl.ds(s0, rows_per_stager)])
        plsc.subcore_barrier()

        i0 = wid * rows_per_worker

        def fire(r, c):
            i = i0 + r
            pltpu.async_copy(
                shared.at[pl.ds(L - 1 - i, L)],
                out.at[pl.ds(pl.multiple_of(i * L, 8), L)],
                sem,
            )
            return c

        def wait_one(r, c):
            pltpu.make_async_copy(
                shared.at[pl.ds(0, L)], out.at[pl.ds(0, L)], sem
            ).wait()
            return c

        def steady(r, c):
            return fire(r, wait_one(r, c))

        c = lax.fori_loop(0, K, fire, 0)
        c = lax.fori_loop(K, rows_per_worker, steady, c)
        lax.fori_loop(0, K, wait_one, c)

    return sc_kernel


@functools.lru_cache(maxsize=None)
def _build_sc_combined_kernel(Vpad, D, L, R, C, LS, KW, KS):
    NCB = L // C
    NRB = LS // R
    W = C + R

    info = plsc.get_sparse_core_info()
    num_workers = info.num_cores * info.num_subcores
    assert NRB * NCB == num_workers
    rows_sp = L - LS
    sp_per_worker = rows_sp // num_workers
    rows_per_stager = Vpad // info.num_subcores

    mesh = plsc.VectorSubcoreMesh(core_axis_name="c", subcore_axis_name="s")

    @functools.partial(
        pl.kernel,
        out_type=jax.ShapeDtypeStruct((L * L, D), jnp.float32),
        name="toeplitz_gather_sc2",
        mesh=mesh,
        scratch_types=[
            pltpu.VMEM((W, D), jnp.float32),
            pltpu.VMEM_SHARED((Vpad, D), jnp.float32),
            pltpu.SemaphoreType.DMA,
            pltpu.SemaphoreType.DMA,
        ],
    )
    def sc_kernel(table, out, win, shared, sem_w, sem_s):
        cid = lax.axis_index("c")
        sid = lax.axis_index("s")
        wid = sid * info.num_cores + cid

        rb = wid // NCB
        cb = wid % NCB
        i0 = rb * R
        j0 = cb * C
        base = pl.multiple_of((L - 1) + j0 - i0 - (R - 1), 8)
        pltpu.sync_copy(table.at[pl.ds(base, W)], win)
        s0 = pl.multiple_of(sid * rows_per_stager, 8)
        pltpu.sync_copy(table.at[pl.ds(s0, rows_per_stager)],
                        shared.at[pl.ds(s0, rows_per_stager)])
        plsc.subcore_barrier()

        g0 = LS + wid * sp_per_worker

        def fire_w(r, c):
            pltpu.async_copy(
                win.at[pl.ds(R - 1 - r, C)],
                out.at[pl.ds(pl.multiple_of((i0 + r) * L + j0, 8), C)],
                sem_w,
            )
            return c

        def wait_w(r, c):
            pltpu.make_async_copy(
                win.at[pl.ds(0, C)], out.at[pl.ds(0, C)], sem_w
            ).wait()
            return c

        def fire_s(r, c):
            i = g0 + r
            pltpu.async_copy(
                shared.at[pl.ds(L - 1 - i, L)],
                out.at[pl.ds(pl.multiple_of(i * L, 8), L)],
                sem_s,
            )
            return c

        def wait_s(r, c):
            pltpu.make_async_copy(
                shared.at[pl.ds(0, L)], out.at[pl.ds(0, L)], sem_s
            ).wait()
            return c

        def steady_w(r, c):
            c = wait_w(r, c)
            return fire_w(r, c)

        def steady_s(r, c):
            c = wait_s(r, c)
            return fire_s(r + KS, c)

        c = lax.fori_loop(0, KW, fire_w, 0)
        c = lax.fori_loop(KW, R, steady_w, c)
        c = lax.fori_loop(0, KW, wait_w, c)
        if sp_per_worker > 0:
            c = lax.fori_loop(0, KS, fire_s, c)
            c = lax.fori_loop(0, sp_per_worker - KS, steady_s, c)
            lax.fori_loop(0, KS, wait_s, c)

    return sc_kernel


def kernel(rel_emb, length):
    V, D = rel_emb.shape
    L = (V + 1) // 2
    table = jnp.concatenate([rel_emb, jnp.zeros((1, D), rel_emb.dtype)], axis=0)
    H = L // 2
    a = _build_sc_kernel_part(V + 1, D, L, 128, 512, 8, 0, H)(table)
    b = _build_sc_kernel_part(V + 1, D, L, 128, 512, 8, H, H)(table)
    return jnp.concatenate([a.reshape(H, L, D), b.reshape(H, L, D)], axis=0)


# speedup vs baseline: 1.7921x; 1.4956x over previous
import functools

import jax
import jax.numpy as jnp
from jax import lax
from jax.experimental import pallas as pl
from jax.experimental.pallas import tpu as pltpu
from jax.experimental.pallas import tpu_sc as plsc


@functools.lru_cache(maxsize=None)
def _build_sc_kernel(Vpad, D, L, R, C, K):
    NCB = L // C
    NRB = L // R
    W = C + R

    info = plsc.get_sparse_core_info()
    num_workers = info.num_cores * info.num_subcores
    assert NRB * NCB == num_workers

    mesh = plsc.VectorSubcoreMesh(core_axis_name="c", subcore_axis_name="s")

    @functools.partial(
        pl.kernel,
        out_type=jax.ShapeDtypeStruct((L * L, D), jnp.float32),
        name="toeplitz_gather_sc",
        mesh=mesh,
        scratch_types=[
            pltpu.VMEM((W, D), jnp.float32),
            pltpu.SemaphoreType.DMA,
        ],
    )
    def sc_kernel(table, out, win, sem):
        wid = lax.axis_index("s") * info.num_cores + lax.axis_index("c")
        rb = wid // NCB
        cb = wid % NCB
        i0 = rb * R
        j0 = cb * C
        base = pl.multiple_of((L - 1) + j0 - i0 - (R - 1), 8)
        pltpu.sync_copy(table.at[pl.ds(base, W)], win)

        def fire(r, c):
            pltpu.async_copy(
                win.at[pl.ds(R - 1 - r, C)],
                out.at[pl.ds(pl.multiple_of((i0 + r) * L + j0, 8), C)],
                sem,
            )
            return c

        def wait_one(r, c):
            pltpu.make_async_copy(
                win.at[pl.ds(0, C)], out.at[pl.ds(0, C)], sem
            ).wait()
            return c

        def steady(r, c):
            return fire(r, wait_one(r, c))

        c = lax.fori_loop(0, K, fire, 0)
        c = lax.fori_loop(K, R, steady, c)
        lax.fori_loop(0, K, wait_one, c)

    return sc_kernel


def kernel(rel_emb, length):
    V, D = rel_emb.shape
    L = (V + 1) // 2
    table = jnp.concatenate([rel_emb, jnp.zeros((1, D), rel_emb.dtype)], axis=0)
    out2d = _build_sc_kernel(V + 1, D, L, 256, 512, 8)(table)
    return out2d.reshape(L, L, D)
